# baseline (speedup 1.0000x reference)
import functools

import jax
import jax.numpy as jnp
from jax import lax
from jax.experimental import pallas as pl
from jax.experimental.pallas import tpu as pltpu
from jax.experimental.pallas import tpu_sc as plsc

Q = 4096
D = 128
N = 100000
BN = 512
NB = 156 + 40
NP = NB * BN
QT = 2048
K = 8
NUM_CLASSES = 1000

_F32_BIG = 3.0e38


def _stage_a_body(x_ref, xb_ref, vals_ref, idx_ref):
    b = pl.program_id(1)
    x = x_ref[...]
    xb = xb_ref[...]
    a2 = jnp.sum(x * x, axis=1, keepdims=True)
    b2 = jnp.sum(xb * xb, axis=1)
    dot = lax.dot_general(x, xb, (((1,), (1,)), ((), ())),
                          preferred_element_type=jnp.float32)
    keys = a2 + b2[None, :] - 2.0 * dot
    lane = lax.broadcasted_iota(jnp.int32, (QT, BN), 1).astype(jnp.float32)
    limit = jnp.float32(N - b * BN)
    keys = jnp.where(lane >= limit, jnp.inf, keys)
    vals_rows = []
    idx_rows = []
    for i in range(K):
        m = jnp.min(keys, axis=1)
        cand = jnp.where(keys == m[:, None], lane, _F32_BIG)
        j = jnp.min(cand, axis=1)
        if i < K - 1:
            keys = jnp.where(lane == j[:, None], jnp.inf, keys)
        vals_rows.append(m)
        idx_rows.append(j + jnp.float32(b * BN))
    vals_ref[0] = jnp.stack(vals_rows)
    idx_ref[0] = jnp.stack(idx_rows)


def _stage_b_body(vals_ref, idx_ref, out_ref):
    vals = vals_ref[...]
    idxs = idx_ref[...]
    rows = []
    for i in range(K):
        m = jnp.min(vals, axis=0)
        hit = vals == m[None, :]
        j = jnp.min(jnp.where(hit, idxs, _F32_BIG), axis=0)
        if i < K - 1:
            vals = jnp.where(hit & (idxs == j[None, :]), jnp.inf, vals)
        rows.append(j)
    out_ref[...] = jnp.stack(rows).astype(jnp.int32)


def _stage_c_body(idx_hbm, y_hbm, out_hbm, idx_v, lab_v, out_v, sem):
    nc = 2
    wid = lax.axis_index("s") * nc + lax.axis_index("c")
    qw = Q // 32
    base = wid * qw
    for i in range(K):
        pltpu.sync_copy(idx_hbm.at[pl.ds(i * Q + base, qw)],
                        idx_v.at[pl.ds(i * qw, qw)])
    for i in range(K):
        pltpu.async_copy(y_hbm.at[idx_v.at[pl.ds(i * qw, qw)]],
                         lab_v.at[pl.ds(i * qw, qw)], sem).wait()
    for g in range(qw // 16):
        labels = [lab_v[pl.ds(i * qw + g * 16, 16)] for i in range(K)]
        zero = jnp.zeros((16,), jnp.int32)
        best = jnp.full((16,), -1, jnp.int32)
        for i in range(K):
            cnt = zero
            for j in range(K):
                cnt = cnt + jnp.where(labels[i] == labels[j], 1, 0)
            score = cnt * 1024 + (1023 - labels[i])
            best = jnp.maximum(best, score)
        pred = 1023 - jnp.bitwise_and(best, 1023)
        out_v[pl.ds(g * 16, 16)] = pred
    pltpu.sync_copy(out_v, out_hbm.at[pl.ds(wid * qw, qw)])


@jax.jit
def kernel(X, X_train, y_train):
    X_train = X_train.astype(jnp.float32)
    X = X.astype(jnp.float32)
    xp = jnp.pad(X_train, ((0, NP - N), (0, 0)))

    vals, idxs = pl.pallas_call(
        _stage_a_body,
        grid=(Q // QT, NB),
        in_specs=[
            pl.BlockSpec((QT, D), lambda q, b: (q, 0)),
            pl.BlockSpec((BN, D), lambda q, b: (b, 0)),
        ],
        out_specs=[
            pl.BlockSpec((1, K, QT), lambda q, b: (b, 0, q)),
            pl.BlockSpec((1, K, QT), lambda q, b: (b, 0, q)),
        ],
        out_shape=[
            jax.ShapeDtypeStruct((NB, K, Q), jnp.float32),
            jax.ShapeDtypeStruct((NB, K, Q), jnp.float32),
        ],
    )(X, xp)

    vals = vals.reshape(NB * K, Q)
    idxs = idxs.reshape(NB * K, Q)

    C = 1024
    knn = pl.pallas_call(
        _stage_b_body,
        grid=(Q // C,),
        in_specs=[
            pl.BlockSpec((NB * K, C), lambda c: (0, c)),
            pl.BlockSpec((NB * K, C), lambda c: (0, c)),
        ],
        out_specs=pl.BlockSpec((K, C), lambda c: (0, c)),
        out_shape=jax.ShapeDtypeStruct((K, Q), jnp.int32),
    )(vals, idxs)

    knn_flat = knn.reshape(K * Q)

    mode_kernel = functools.partial(
        pl.kernel,
        mesh=plsc.VectorSubcoreMesh(core_axis_name="c", subcore_axis_name="s"),
        out_type=jax.ShapeDtypeStruct((Q,), jnp.int32),
        scratch_types=[
            pltpu.VMEM((Q * K // 32,), jnp.int32),
            pltpu.VMEM((Q * K // 32,), jnp.int32),
            pltpu.VMEM((Q // 32,), jnp.int32),
            pltpu.SemaphoreType.DMA,
        ],
    )(_stage_c_body)
    y_pred = mode_kernel(knn_flat, y_train.astype(jnp.int32))
    return y_pred


# speedup vs baseline: 1.0240x; 1.0240x over previous
import functools

import jax
import jax.numpy as jnp
from jax import lax
from jax.experimental import pallas as pl
from jax.experimental.pallas import tpu as pltpu
from jax.experimental.pallas import tpu_sc as plsc

Q = 4096
D = 128
N = 100000
BN = 512
NB = 156 + 40
NP = NB * BN
QT = 2048
K = 8
NUM_CLASSES = 1000

_F32_BIG = 3.0e38


def _stage_a_body(x_ref, xt_ref, vals_ref, idx_ref):
    b = pl.program_id(1)
    x = x_ref[...]
    xt = xt_ref[...]
    a2 = jnp.sum(x * x, axis=1, keepdims=True)
    b2 = jnp.sum(xt * xt, axis=0, keepdims=True)
    dot = lax.dot_general(x, xt, (((1,), (0,)), ((), ())),
                          preferred_element_type=jnp.float32)
    keys = a2 + b2 - 2.0 * dot
    lane = lax.broadcasted_iota(jnp.int32, (QT, BN), 1).astype(jnp.float32)
    limit = jnp.float32(N - b * BN)
    keys = jnp.where(lane >= limit, jnp.inf, keys)
    vals_rows = []
    idx_rows = []
    for i in range(K):
        m = jnp.min(keys, axis=1)
        cand = jnp.where(keys == m[:, None], lane, _F32_BIG)
        j = jnp.min(cand, axis=1)
        if i < K - 1:
            keys = jnp.where(lane == j[:, None], jnp.inf, keys)
        vals_rows.append(m)
        idx_rows.append(j + jnp.float32(b * BN))
    vals_ref[0] = jnp.stack(vals_rows)
    idx_ref[0] = jnp.stack(idx_rows)


def _stage_b_body(vals_ref, idx_ref, out_ref):
    vals = vals_ref[...]
    idxs = idx_ref[...]
    rows = []
    for i in range(K):
        m = jnp.min(vals, axis=0)
        hit = vals == m[None, :]
        j = jnp.min(jnp.where(hit, idxs, _F32_BIG), axis=0)
        if i < K - 1:
            vals = jnp.where(hit & (idxs == j[None, :]), jnp.inf, vals)
        rows.append(j)
    out_ref[...] = jnp.stack(rows).astype(jnp.int32)


def _stage_c_body(idx_hbm, y_hbm, out_hbm, idx_v, lab_v, out_v, sem):
    nc = 2
    wid = lax.axis_index("s") * nc + lax.axis_index("c")
    qw = Q // 32
    base = wid * qw
    for i in range(K):
        pltpu.sync_copy(idx_hbm.at[pl.ds(i * Q + base, qw)],
                        idx_v.at[pl.ds(i * qw, qw)])
    for i in range(K):
        pltpu.async_copy(y_hbm.at[idx_v.at[pl.ds(i * qw, qw)]],
                         lab_v.at[pl.ds(i * qw, qw)], sem).wait()
    for g in range(qw // 16):
        labels = [lab_v[pl.ds(i * qw + g * 16, 16)] for i in range(K)]
        zero = jnp.zeros((16,), jnp.int32)
        best = jnp.full((16,), -1, jnp.int32)
        for i in range(K):
            cnt = zero
            for j in range(K):
                cnt = cnt + jnp.where(labels[i] == labels[j], 1, 0)
            score = cnt * 1024 + (1023 - labels[i])
            best = jnp.maximum(best, score)
        pred = 1023 - jnp.bitwise_and(best, 1023)
        out_v[pl.ds(g * 16, 16)] = pred
    pltpu.sync_copy(out_v, out_hbm.at[pl.ds(wid * qw, qw)])


@jax.jit
def kernel(X, X_train, y_train):
    X_train = X_train.astype(jnp.float32)
    X = X.astype(jnp.float32)
    xt = jnp.pad(X_train, ((0, NP - N), (0, 0))).T

    vals, idxs = pl.pallas_call(
        _stage_a_body,
        grid=(Q // QT, NB),
        in_specs=[
            pl.BlockSpec((QT, D), lambda q, b: (q, 0)),
            pl.BlockSpec((D, BN), lambda q, b: (0, b)),
        ],
        out_specs=[
            pl.BlockSpec((1, K, QT), lambda q, b: (b, 0, q)),
            pl.BlockSpec((1, K, QT), lambda q, b: (b, 0, q)),
        ],
        out_shape=[
            jax.ShapeDtypeStruct((NB, K, Q), jnp.float32),
            jax.ShapeDtypeStruct((NB, K, Q), jnp.float32),
        ],
    )(X, xt)

    vals = vals.reshape(NB * K, Q)
    idxs = idxs.reshape(NB * K, Q)

    C = 1024
    knn = pl.pallas_call(
        _stage_b_body,
        grid=(Q // C,),
        in_specs=[
            pl.BlockSpec((NB * K, C), lambda c: (0, c)),
            pl.BlockSpec((NB * K, C), lambda c: (0, c)),
        ],
        out_specs=pl.BlockSpec((K, C), lambda c: (0, c)),
        out_shape=jax.ShapeDtypeStruct((K, Q), jnp.int32),
    )(vals, idxs)

    knn_flat = knn.reshape(K * Q)

    mode_kernel = functools.partial(
        pl.kernel,
        mesh=plsc.VectorSubcoreMesh(core_axis_name="c", subcore_axis_name="s"),
        out_type=jax.ShapeDtypeStruct((Q,), jnp.int32),
        scratch_types=[
            pltpu.VMEM((Q * K // 32,), jnp.int32),
            pltpu.VMEM((Q * K // 32,), jnp.int32),
            pltpu.VMEM((Q // 32,), jnp.int32),
            pltpu.SemaphoreType.DMA,
        ],
    )(_stage_c_body)
    y_pred = mode_kernel(knn_flat, y_train.astype(jnp.int32))
    return y_pred


# speedup vs baseline: 1.1388x; 1.1121x over previous
import functools

import jax
import jax.numpy as jnp
from jax import lax
from jax.experimental import pallas as pl
from jax.experimental.pallas import tpu as pltpu
from jax.experimental.pallas import tpu_sc as plsc

Q = 4096
D = 128
N = 100000
BN = 1024
NB = 98
NP = NB * BN
QT = 2048
K = 8
NUM_CLASSES = 1000

_F32_BIG = 3.0e38


def _stage_a_body(x_ref, xt_ref, vals_ref, idx_ref):
    b = pl.program_id(1)
    x = x_ref[...]
    xt = xt_ref[...]
    a2 = jnp.sum(x * x, axis=1, keepdims=True)
    b2 = jnp.sum(xt * xt, axis=0, keepdims=True)
    dot = lax.dot_general(x, xt, (((1,), (0,)), ((), ())),
                          preferred_element_type=jnp.float32)
    keys = a2 + b2 - 2.0 * dot
    lane = lax.broadcasted_iota(jnp.int32, (QT, BN), 1).astype(jnp.float32)
    limit = jnp.float32(N - b * BN)
    keys = jnp.where(lane >= limit, jnp.inf, keys)
    vals_rows = []
    idx_rows = []
    for i in range(K):
        m = jnp.min(keys, axis=1)
        cand = jnp.where(keys == m[:, None], lane, _F32_BIG)
        j = jnp.min(cand, axis=1)
        if i < K - 1:
            keys = jnp.where(lane == j[:, None], jnp.inf, keys)
        vals_rows.append(m)
        idx_rows.append(j + jnp.float32(b * BN))
    vals_ref[0] = jnp.stack(vals_rows)
    idx_ref[0] = jnp.stack(idx_rows)


def _stage_b_body(vals_ref, idx_ref, out_ref):
    vals = vals_ref[...]
    idxs = idx_ref[...]
    rows = []
    for i in range(K):
        m = jnp.min(vals, axis=0)
        hit = vals == m[None, :]
        j = jnp.min(jnp.where(hit, idxs, _F32_BIG), axis=0)
        if i < K - 1:
            vals = jnp.where(hit & (idxs == j[None, :]), jnp.inf, vals)
        rows.append(j)
    out_ref[...] = jnp.stack(rows).astype(jnp.int32)


def _stage_c_body(idx_hbm, y_hbm, out_hbm, idx_v, lab_v, out_v, sem):
    nc = 2
    wid = lax.axis_index("s") * nc + lax.axis_index("c")
    qw = Q // 32
    base = wid * qw
    for i in range(K):
        pltpu.sync_copy(idx_hbm.at[pl.ds(i * Q + base, qw)],
                        idx_v.at[pl.ds(i * qw, qw)])
    for i in range(K):
        pltpu.async_copy(y_hbm.at[idx_v.at[pl.ds(i * qw, qw)]],
                         lab_v.at[pl.ds(i * qw, qw)], sem).wait()
    for g in range(qw // 16):
        labels = [lab_v[pl.ds(i * qw + g * 16, 16)] for i in range(K)]
        zero = jnp.zeros((16,), jnp.int32)
        best = jnp.full((16,), -1, jnp.int32)
        for i in range(K):
            cnt = zero
            for j in range(K):
                cnt = cnt + jnp.where(labels[i] == labels[j], 1, 0)
            score = cnt * 1024 + (1023 - labels[i])
            best = jnp.maximum(best, score)
        pred = 1023 - jnp.bitwise_and(best, 1023)
        out_v[pl.ds(g * 16, 16)] = pred
    pltpu.sync_copy(out_v, out_hbm.at[pl.ds(wid * qw, qw)])


@jax.jit
def kernel(X, X_train, y_train):
    X_train = X_train.astype(jnp.float32)
    X = X.astype(jnp.float32)
    xt = jnp.pad(X_train, ((0, NP - N), (0, 0))).T

    vals, idxs = pl.pallas_call(
        _stage_a_body,
        grid=(Q // QT, NB),
        in_specs=[
            pl.BlockSpec((QT, D), lambda q, b: (q, 0)),
            pl.BlockSpec((D, BN), lambda q, b: (0, b)),
        ],
        out_specs=[
            pl.BlockSpec((1, K, QT), lambda q, b: (b, 0, q)),
            pl.BlockSpec((1, K, QT), lambda q, b: (b, 0, q)),
        ],
        out_shape=[
            jax.ShapeDtypeStruct((NB, K, Q), jnp.float32),
            jax.ShapeDtypeStruct((NB, K, Q), jnp.float32),
        ],
    )(X, xt)

    vals = vals.reshape(NB * K, Q)
    idxs = idxs.reshape(NB * K, Q)

    C = 1024
    knn = pl.pallas_call(
        _stage_b_body,
        grid=(Q // C,),
        in_specs=[
            pl.BlockSpec((NB * K, C), lambda c: (0, c)),
            pl.BlockSpec((NB * K, C), lambda c: (0, c)),
        ],
        out_specs=pl.BlockSpec((K, C), lambda c: (0, c)),
        out_shape=jax.ShapeDtypeStruct((K, Q), jnp.int32),
    )(vals, idxs)

    knn_flat = knn.reshape(K * Q)

    mode_kernel = functools.partial(
        pl.kernel,
        mesh=plsc.VectorSubcoreMesh(core_axis_name="c", subcore_axis_name="s"),
        out_type=jax.ShapeDtypeStruct((Q,), jnp.int32),
        scratch_types=[
            pltpu.VMEM((Q * K // 32,), jnp.int32),
            pltpu.VMEM((Q * K // 32,), jnp.int32),
            pltpu.VMEM((Q // 32,), jnp.int32),
            pltpu.SemaphoreType.DMA,
        ],
    )(_stage_c_body)
    y_pred = mode_kernel(knn_flat, y_train.astype(jnp.int32))
    return y_pred


# speedup vs baseline: 1.1681x; 1.0257x over previous
import functools

import jax
import jax.numpy as jnp
from jax import lax
from jax.experimental import pallas as pl
from jax.experimental.pallas import tpu as pltpu
from jax.experimental.pallas import tpu_sc as plsc

Q = 4096
D = 128
N = 100000
BN = 2048
NB = 49
NP = NB * BN
QT = 2048
K = 8
NUM_CLASSES = 1000

_F32_BIG = 3.0e38


def _stage_a_body(x_ref, xt_ref, vals_ref, idx_ref):
    b = pl.program_id(1)
    x = x_ref[...]
    xt = xt_ref[...]
    a2 = jnp.sum(x * x, axis=1, keepdims=True)
    b2 = jnp.sum(xt * xt, axis=0, keepdims=True)
    dot = lax.dot_general(x, xt, (((1,), (0,)), ((), ())),
                          preferred_element_type=jnp.float32)
    keys = a2 + b2 - 2.0 * dot
    lane = lax.broadcasted_iota(jnp.int32, (QT, BN), 1).astype(jnp.float32)
    limit = jnp.float32(N - b * BN)
    keys = jnp.where(lane >= limit, jnp.inf, keys)
    vals_rows = []
    idx_rows = []
    for i in range(K):
        m = jnp.min(keys, axis=1)
        cand = jnp.where(keys == m[:, None], lane, _F32_BIG)
        j = jnp.min(cand, axis=1)
        if i < K - 1:
            keys = jnp.where(lane == j[:, None], jnp.inf, keys)
        vals_rows.append(m)
        idx_rows.append(j + jnp.float32(b * BN))
    vals_ref[0] = jnp.stack(vals_rows)
    idx_ref[0] = jnp.stack(idx_rows)


def _stage_b_body(vals_ref, idx_ref, out_ref):
    vals = vals_ref[...]
    idxs = idx_ref[...]
    rows = []
    for i in range(K):
        m = jnp.min(vals, axis=0)
        hit = vals == m[None, :]
        j = jnp.min(jnp.where(hit, idxs, _F32_BIG), axis=0)
        if i < K - 1:
            vals = jnp.where(hit & (idxs == j[None, :]), jnp.inf, vals)
        rows.append(j)
    out_ref[...] = jnp.stack(rows).astype(jnp.int32)


def _stage_c_body(idx_hbm, y_hbm, out_hbm, idx_v, lab_v, out_v, sem):
    nc = 2
    wid = lax.axis_index("s") * nc + lax.axis_index("c")
    qw = Q // 32
    base = wid * qw
    for i in range(K):
        pltpu.sync_copy(idx_hbm.at[pl.ds(i * Q + base, qw)],
                        idx_v.at[pl.ds(i * qw, qw)])
    for i in range(K):
        pltpu.async_copy(y_hbm.at[idx_v.at[pl.ds(i * qw, qw)]],
                         lab_v.at[pl.ds(i * qw, qw)], sem).wait()
    for g in range(qw // 16):
        labels = [lab_v[pl.ds(i * qw + g * 16, 16)] for i in range(K)]
        zero = jnp.zeros((16,), jnp.int32)
        best = jnp.full((16,), -1, jnp.int32)
        for i in range(K):
            cnt = zero
            for j in range(K):
                cnt = cnt + jnp.where(labels[i] == labels[j], 1, 0)
            score = cnt * 1024 + (1023 - labels[i])
            best = jnp.maximum(best, score)
        pred = 1023 - jnp.bitwise_and(best, 1023)
        out_v[pl.ds(g * 16, 16)] = pred
    pltpu.sync_copy(out_v, out_hbm.at[pl.ds(wid * qw, qw)])


@jax.jit
def kernel(X, X_train, y_train):
    X_train = X_train.astype(jnp.float32)
    X = X.astype(jnp.float32)
    xt = jnp.pad(X_train, ((0, NP - N), (0, 0))).T

    vals, idxs = pl.pallas_call(
        _stage_a_body,
        grid=(Q // QT, NB),
        in_specs=[
            pl.BlockSpec((QT, D), lambda q, b: (q, 0)),
            pl.BlockSpec((D, BN), lambda q, b: (0, b)),
        ],
        out_specs=[
            pl.BlockSpec((1, K, QT), lambda q, b: (b, 0, q)),
            pl.BlockSpec((1, K, QT), lambda q, b: (b, 0, q)),
        ],
        out_shape=[
            jax.ShapeDtypeStruct((NB, K, Q), jnp.float32),
            jax.ShapeDtypeStruct((NB, K, Q), jnp.float32),
        ],
    )(X, xt)

    vals = vals.reshape(NB * K, Q)
    idxs = idxs.reshape(NB * K, Q)

    C = 1024
    knn = pl.pallas_call(
        _stage_b_body,
        grid=(Q // C,),
        in_specs=[
            pl.BlockSpec((NB * K, C), lambda c: (0, c)),
            pl.BlockSpec((NB * K, C), lambda c: (0, c)),
        ],
        out_specs=pl.BlockSpec((K, C), lambda c: (0, c)),
        out_shape=jax.ShapeDtypeStruct((K, Q), jnp.int32),
    )(vals, idxs)

    knn_flat = knn.reshape(K * Q)

    mode_kernel = functools.partial(
        pl.kernel,
        mesh=plsc.VectorSubcoreMesh(core_axis_name="c", subcore_axis_name="s"),
        out_type=jax.ShapeDtypeStruct((Q,), jnp.int32),
        scratch_types=[
            pltpu.VMEM((Q * K // 32,), jnp.int32),
            pltpu.VMEM((Q * K // 32,), jnp.int32),
            pltpu.VMEM((Q // 32,), jnp.int32),
            pltpu.SemaphoreType.DMA,
        ],
    )(_stage_c_body)
    y_pred = mode_kernel(knn_flat, y_train.astype(jnp.int32))
    return y_pred
